# baseline (speedup 1.0000x reference)
import functools
import math

import jax
import jax.numpy as jnp
from jax import lax
from jax.experimental import pallas as pl
from jax.experimental.pallas import tpu as pltpu
from jax.experimental.pallas import tpu_sc as plsc

D_MODEL = 128
SCALE = math.sqrt(float(D_MODEL))
LANES = 16
NUM_WORKERS = 32
CHUNK = 128
NBUF = 5
LEAD = 3


def _emb_body(x_hbm, table_hbm, out_hbm, idx_all, *scratch, bpw, n_chunks):
    rows = scratch[:NBUF]
    sg = scratch[NBUF:2 * NBUF]
    sw = scratch[2 * NBUF:]
    wid = lax.axis_index("s") * 2 + lax.axis_index("c")
    base = wid * bpw

    pltpu.sync_copy(x_hbm.at[pl.ds(base, bpw)], idx_all)

    def start_gather(ci, b):
        pltpu.async_copy(table_hbm.at[idx_all.at[pl.ds(ci * CHUNK, CHUNK)]],
                         rows[b], sg[b])

    def wait_gather(b):
        pltpu.make_async_copy(table_hbm.at[idx_all.at[pl.ds(0, CHUNK)]],
                              rows[b], sg[b]).wait()

    def start_writeback(ci, b):
        pltpu.async_copy(rows[b], out_hbm.at[pl.ds(base + ci * CHUNK, CHUNK)],
                         sw[b])

    def wait_writeback(b):
        pltpu.make_async_copy(rows[b], out_hbm.at[pl.ds(base, CHUNK)],
                              sw[b]).wait()

    for ci in range(LEAD):
        start_gather(ci, ci)

    def outer(g, carry):
        for b in range(NBUF):
            ci = NBUF * g + b
            b2 = (b + LEAD) % NBUF
            wait_gather(b)

            @pl.when(ci + LEAD < n_chunks)
            def _():
                @pl.when(ci >= NBUF - LEAD)
                def _():
                    wait_writeback(b2)
                start_gather(ci + LEAD, b2)

            def scale_row(r, c):
                for k in range(D_MODEL // LANES):
                    sl = pl.ds(k * LANES, LANES)
                    rows[b][r, sl] = rows[b][r, sl] * SCALE
                return c

            lax.fori_loop(0, CHUNK, scale_row, 0)
            start_writeback(ci, b)
        return carry

    lax.fori_loop(0, n_chunks // NBUF, outer, 0)
    for b in range(NBUF):
        wait_writeback(b)


@functools.partial(jax.jit, static_argnames=())
def kernel(x, table):
    b, h = x.shape
    n = b * h
    x_flat = x.reshape(n).astype(jnp.int32)
    bpw = n // NUM_WORKERS
    n_chunks = bpw // CHUNK

    mesh = plsc.VectorSubcoreMesh(core_axis_name="c", subcore_axis_name="s")
    grid_kernel = pl.kernel(
        functools.partial(_emb_body, bpw=bpw, n_chunks=n_chunks),
        out_type=jax.ShapeDtypeStruct((n, D_MODEL), jnp.float32),
        mesh=mesh,
        scratch_types=(
            [pltpu.VMEM((bpw,), jnp.int32)]
            + [pltpu.VMEM((CHUNK, D_MODEL), jnp.float32) for _ in range(NBUF)]
            + [pltpu.SemaphoreType.DMA for _ in range(2 * NBUF)]
        ),
    )
    out = grid_kernel(x_flat, table)
    return out.reshape(b, h, D_MODEL)
